# baseline (speedup 1.0000x reference)
import functools

import jax
import jax.numpy as jnp
import numpy as np
from jax import lax
from jax.experimental import pallas as pl
from jax.experimental.pallas import tpu as pltpu
from jax.experimental.pallas import tpu_sc as plsc

_N = 100000
_D = 64
_B = 4096
_NC = 2
_NS = 16
_NW = _NC * _NS
_PER_W = 3200
_CHUNK = 128
_NCHUNK = _PER_W // _CHUNK
_NPAD = _NW * _PER_W
_BPW = _B // _NW

_U32 = np.uint32


def _rotl(x, r):
    return (x << _U32(r)) | (x >> _U32(32 - r))


def _threefry2x32_raw(k1, k2, x1, x2):
    x = [x1.astype(_U32).copy(), x2.astype(_U32).copy()]
    rotations = [(13, 15, 26, 6), (17, 29, 16, 24)]
    ks = [_U32(k1), _U32(k2), _U32(k1) ^ _U32(k2) ^ _U32(0x1BD11BDA)]
    with np.errstate(over="ignore"):
        x[0] = x[0] + ks[0]
        x[1] = x[1] + ks[1]
        for i in range(5):
            for r in rotations[i % 2]:
                x[0] = x[0] + x[1]
                x[1] = _rotl(x[1], r)
                x[1] = x[0] ^ x[1]
            x[0] = x[0] + ks[(i + 1) % 3]
            x[1] = x[1] + ks[(i + 2) % 3] + _U32(i + 1)
    return x[0], x[1]


def _key_split(key):
    b1, b2 = _threefry2x32_raw(key[0], key[1],
                               np.zeros(2, _U32), np.arange(2, dtype=_U32))
    return np.stack([b1, b2], axis=1)


def _permutation_np(n):
    key = _key_split(np.zeros(2, _U32))[0]
    x = np.arange(n, dtype=np.int32)
    num_rounds = int(np.ceil(3 * np.log(max(1, n)) / np.log(np.iinfo(_U32).max)))
    for _ in range(num_rounds):
        key, subkey = _key_split(key)
        b1, b2 = _threefry2x32_raw(subkey[0], subkey[1],
                                   np.zeros(n, _U32), np.arange(n, dtype=_U32))
        x = x[np.argsort(b1 ^ b2, kind="stable")]
    return x


_SIGMA = _permutation_np(_N)


@functools.partial(
    pl.kernel,
    out_type=(
        jax.ShapeDtypeStruct((_NPAD,), jnp.int32),
        jax.ShapeDtypeStruct((_B,), jnp.int32),
    ),
    mesh=plsc.VectorSubcoreMesh(core_axis_name="c", subcore_axis_name="s"),
    compiler_params=pltpu.CompilerParams(use_tc_tiling_on_sc=False,
                                         needs_layout_passes=False),
    scratch_types=[
        pltpu.VMEM((_N,), jnp.int32),
        pltpu.VMEM((_PER_W,), jnp.int32),
        pltpu.VMEM((_PER_W,), jnp.int32),
        pltpu.VMEM((_BPW,), jnp.int32),
        pltpu.VMEM((_BPW,), jnp.int32),
    ],
)
def _shuffle_ids(indices_hbm, sigma_hbm, bsig_hbm,
                 ind2_out, bidx_out,
                 indfull_v, sig_v, ind2_v, bsig_v, bidx_v):
    wid = lax.axis_index("s") * _NC + lax.axis_index("c")
    abase = wid * _PER_W
    bbase = wid * _BPW

    pltpu.sync_copy(sigma_hbm.at[pl.ds(abase, _PER_W)], sig_v)
    pltpu.sync_copy(bsig_hbm.at[pl.ds(bbase, _BPW)], bsig_v)
    pltpu.sync_copy(indices_hbm, indfull_v)

    def _permute(k, _):
        sl = pl.ds(k * 16, 16)
        ind2_v[sl] = plsc.load_gather(indfull_v, [sig_v[sl]])
        return 0

    lax.fori_loop(0, _PER_W // 16, _permute, 0)

    for j in range(_BPW // 16):
        sl = pl.ds(j * 16, 16)
        bidx_v[sl] = plsc.load_gather(indfull_v, [bsig_v[sl]])

    pltpu.sync_copy(bidx_v, bidx_out.at[pl.ds(bbase, _BPW)])
    pltpu.sync_copy(ind2_v, ind2_out.at[pl.ds(abase, _PER_W)])


@functools.partial(
    pl.kernel,
    out_type=jax.ShapeDtypeStruct((_B, _D), jnp.float32),
    mesh=plsc.VectorSubcoreMesh(core_axis_name="c", subcore_axis_name="s"),
    compiler_params=pltpu.CompilerParams(use_tc_tiling_on_sc=False),
    scratch_types=[
        pltpu.VMEM((_BPW,), jnp.int32),
        pltpu.VMEM((_BPW, _D), jnp.float32),
        pltpu.SemaphoreType.DMA,
    ],
)
def _row_gather(table_hbm, bidx_hbm, batch_out, bidx_v, rows_v, sem):
    wid = lax.axis_index("s") * _NC + lax.axis_index("c")
    bbase = wid * _BPW
    pltpu.sync_copy(bidx_hbm.at[pl.ds(bbase, _BPW)], bidx_v)
    pltpu.async_copy(table_hbm.at[bidx_v], rows_v, sem).wait()
    pltpu.sync_copy(rows_v, batch_out.at[pl.ds(bbase, _BPW)])


def kernel(table, indices, position, reset):
    reset_b = reset[0]
    sig = jnp.where(reset_b, jnp.asarray(_SIGMA), lax.iota(jnp.int32, _N))
    pos2 = jnp.where(reset_b, jnp.int32(0), position)
    bsig = lax.dynamic_slice_in_dim(sig, pos2, _B)
    sig_pad = jnp.concatenate([sig, jnp.zeros((_NPAD - _N,), jnp.int32)])
    ind2_pad, bidx = _shuffle_ids(indices, sig_pad, bsig)
    batch = _row_gather(table, bidx)
    indices2 = lax.slice_in_dim(ind2_pad, 0, _N)
    new_position = pos2 + _B
    reset_condition = pos2 >= _N
    return batch, indices2, new_position, reset_condition


# speedup vs baseline: 1.1022x; 1.1022x over previous
import functools

import jax
import jax.numpy as jnp
import numpy as np
from jax import lax
from jax.experimental import pallas as pl
from jax.experimental.pallas import tpu as pltpu
from jax.experimental.pallas import tpu_sc as plsc

_N = 100000
_D = 64
_B = 4096
_NC = 2
_NS = 16
_NW = _NC * _NS
_PER_W = 3200
_CHUNK = 128
_NCHUNK = _PER_W // _CHUNK
_NPAD = _NW * _PER_W
_BPW = _B // _NW

_U32 = np.uint32


def _rotl(x, r):
    return (x << _U32(r)) | (x >> _U32(32 - r))


def _threefry2x32_raw(k1, k2, x1, x2):
    x = [x1.astype(_U32).copy(), x2.astype(_U32).copy()]
    rotations = [(13, 15, 26, 6), (17, 29, 16, 24)]
    ks = [_U32(k1), _U32(k2), _U32(k1) ^ _U32(k2) ^ _U32(0x1BD11BDA)]
    with np.errstate(over="ignore"):
        x[0] = x[0] + ks[0]
        x[1] = x[1] + ks[1]
        for i in range(5):
            for r in rotations[i % 2]:
                x[0] = x[0] + x[1]
                x[1] = _rotl(x[1], r)
                x[1] = x[0] ^ x[1]
            x[0] = x[0] + ks[(i + 1) % 3]
            x[1] = x[1] + ks[(i + 2) % 3] + _U32(i + 1)
    return x[0], x[1]


def _key_split(key):
    b1, b2 = _threefry2x32_raw(key[0], key[1],
                               np.zeros(2, _U32), np.arange(2, dtype=_U32))
    return np.stack([b1, b2], axis=1)


def _permutation_np(n):
    key = _key_split(np.zeros(2, _U32))[0]
    x = np.arange(n, dtype=np.int32)
    num_rounds = int(np.ceil(3 * np.log(max(1, n)) / np.log(np.iinfo(_U32).max)))
    for _ in range(num_rounds):
        key, subkey = _key_split(key)
        b1, b2 = _threefry2x32_raw(subkey[0], subkey[1],
                                   np.zeros(n, _U32), np.arange(n, dtype=_U32))
        x = x[np.argsort(b1 ^ b2, kind="stable")]
    return x


_SIGMA = _permutation_np(_N)


@functools.partial(
    pl.kernel,
    out_type=(
        jax.ShapeDtypeStruct((_NPAD,), jnp.int32),
        jax.ShapeDtypeStruct((_B,), jnp.int32),
    ),
    mesh=plsc.VectorSubcoreMesh(core_axis_name="c", subcore_axis_name="s"),
    compiler_params=pltpu.CompilerParams(use_tc_tiling_on_sc=False,
                                         needs_layout_passes=False),
    scratch_types=[
        pltpu.VMEM_SHARED((_N,), jnp.int32),
        pltpu.VMEM((_PER_W,), jnp.int32),
        pltpu.VMEM((_PER_W,), jnp.int32),
        pltpu.VMEM((_BPW,), jnp.int32),
        pltpu.VMEM((_BPW,), jnp.int32),
        pltpu.SemaphoreType.DMA,
        pltpu.SemaphoreType.DMA,
    ],
)
def _shuffle_ids(indices_hbm, sigma_hbm, bsig_hbm,
                 ind2_out, bidx_out,
                 indsh_v, sig_v, ind2_v, bsig_v, bidx_v, sem_a, sem_b):
    wid = lax.axis_index("s") * _NC + lax.axis_index("c")
    sid = lax.axis_index("s")
    abase = wid * _PER_W
    bbase = wid * _BPW

    pltpu.sync_copy(sigma_hbm.at[pl.ds(abase, _PER_W)], sig_v)
    pltpu.sync_copy(bsig_hbm.at[pl.ds(bbase, _BPW)], bsig_v)

    @pl.when(sid == 0)
    def _stage():
        pltpu.sync_copy(indices_hbm, indsh_v)

    plsc.subcore_barrier()

    copies = []
    for j in range(_NCHUNK):
        sl = pl.ds(j * _CHUNK, _CHUNK)
        copies.append(
            pltpu.make_async_copy(indsh_v.at[sig_v.at[sl]],
                                  ind2_v.at[sl], sem_a))
        copies[-1].start()

    pltpu.async_copy(indsh_v.at[bsig_v], bidx_v, sem_b).wait()
    pltpu.sync_copy(bidx_v, bidx_out.at[pl.ds(bbase, _BPW)])

    for c in copies:
        c.wait()
    pltpu.sync_copy(ind2_v, ind2_out.at[pl.ds(abase, _PER_W)])


@functools.partial(
    pl.kernel,
    out_type=jax.ShapeDtypeStruct((_B, _D), jnp.float32),
    mesh=plsc.VectorSubcoreMesh(core_axis_name="c", subcore_axis_name="s"),
    compiler_params=pltpu.CompilerParams(use_tc_tiling_on_sc=False),
    scratch_types=[
        pltpu.VMEM((_BPW,), jnp.int32),
        pltpu.VMEM((_BPW, _D), jnp.float32),
        pltpu.SemaphoreType.DMA,
    ],
)
def _row_gather(table_hbm, bidx_hbm, batch_out, bidx_v, rows_v, sem):
    wid = lax.axis_index("s") * _NC + lax.axis_index("c")
    bbase = wid * _BPW
    pltpu.sync_copy(bidx_hbm.at[pl.ds(bbase, _BPW)], bidx_v)
    pltpu.async_copy(table_hbm.at[bidx_v], rows_v, sem).wait()
    pltpu.sync_copy(rows_v, batch_out.at[pl.ds(bbase, _BPW)])


def kernel(table, indices, position, reset):
    reset_b = reset[0]
    sig = jnp.where(reset_b, jnp.asarray(_SIGMA), lax.iota(jnp.int32, _N))
    pos2 = jnp.where(reset_b, jnp.int32(0), position)
    bsig = lax.dynamic_slice_in_dim(sig, pos2, _B)
    sig_pad = jnp.concatenate([sig, jnp.zeros((_NPAD - _N,), jnp.int32)])
    ind2_pad, bidx = _shuffle_ids(indices, sig_pad, bsig)
    batch = _row_gather(table, bidx)
    indices2 = lax.slice_in_dim(ind2_pad, 0, _N)
    new_position = pos2 + _B
    reset_condition = pos2 >= _N
    return batch, indices2, new_position, reset_condition
